# baseline (speedup 1.0000x reference)
import math

import jax
import jax.numpy as jnp
from jax.experimental import pallas as pl
from jax.experimental.pallas import tpu as pltpu

B, N, OBS, HID, QK = 16, 128, 128, 64, 32
KTOP = N // 4
MB = 16
_INV_SQRT_QK = 1.0 / math.sqrt(QK)
_F32 = jnp.float32


def _hyper_body(hs_ref, We_ref, be_ref, Wq_ref, bq_ref, W1_ref, b1_ref,
                W2_ref, b2_ref, Wa1_ref, ba1_ref, Wa2_ref, ba2_ref,
                We1_ref, be1_ref, We2_ref, be2_ref, out_ref, H_ref):
    hsf = hs_ref[...].reshape(MB * N, OBS)

    aff = jnp.maximum(jnp.dot(hsf, We_ref[...], preferred_element_type=_F32)
                      + be_ref[...], 0.0)
    qf = jnp.dot(hsf, Wq_ref[...], preferred_element_type=_F32) + bq_ref[...]
    S3 = jnp.stack([
        jax.lax.dot_general(qf[b * N:(b + 1) * N], qf[b * N:(b + 1) * N],
                            (((1,), (1,)), ((), ())),
                            preferred_element_type=_F32)
        for b in range(MB)]) * _INV_SQRT_QK

    lane_iota = jax.lax.broadcasted_iota(jnp.int32, (1, 1, N), 2)
    Ss = S3
    for sz_log in range(1, 8):
        sz = 1 << sz_log
        for st_log in range(sz_log - 1, -1, -1):
            st = 1 << st_log
            lower_m = (lane_iota & st) == 0
            keep_m = jnp.logical_not(
                jnp.logical_xor(lower_m, (lane_iota & sz) == 0))
            partner = jnp.where(lower_m, jnp.roll(Ss, -st, axis=2),
                                jnp.roll(Ss, st, axis=2))
            Ss = jnp.where(keep_m, jnp.minimum(Ss, partner),
                           jnp.maximum(Ss, partner))
    thr = Ss[:, :, N - KTOP:N - KTOP + 1]
    gtf = (S3 > thr).astype(_F32).reshape(MB * N, N)
    eqf = (S3 == thr).astype(_F32).reshape(MB * N, N)
    col_i = jax.lax.broadcasted_iota(jnp.int32, (N, N), 1)
    ones_nn = jnp.ones((N, N), _F32)
    lt_mat = (jax.lax.broadcasted_iota(jnp.int32, (N, N), 0)
              <= col_i).astype(_F32)
    cntf = (jnp.dot(gtf, ones_nn, preferred_element_type=_F32)
            + jnp.dot(eqf, lt_mat, preferred_element_type=_F32))
    Hf = jnp.where((gtf > 0.0) | ((eqf > 0.0) & (cntf <= float(KTOP))),
                   1.0, 0.0)
    H3 = Hf.reshape(MB, N, N)

    xf = jnp.maximum(jnp.dot(aff, W1_ref[...], preferred_element_type=_F32)
                     + b1_ref[...], 0.0)
    xf = jnp.maximum(jnp.dot(xf, W2_ref[...], preferred_element_type=_F32)
                     + b2_ref[...], 0.0)

    e0f = jnp.concatenate([
        jnp.dot(H3[b], xf[b * N:(b + 1) * N], preferred_element_type=_F32)
        for b in range(MB)], axis=0)

    Wa1 = Wa1_ref[...]
    eaf = (jnp.dot(e0f, Wa1[HID:], preferred_element_type=_F32)
           + ba1_ref[...])
    ea3 = eaf.reshape(MB, N, 32)
    xaT3 = jnp.stack([
        jax.lax.dot_general(Wa1[:HID], xf[b * N:(b + 1) * N],
                            (((0,), (1,)), ((), ())),
                            preferred_element_type=_F32)
        for b in range(MB)])

    w2 = Wa2_ref[...]
    EC = 16
    chunks = []
    for e0c in range(0, N, EC):
        ea_c = ea3[:, e0c:e0c + EC, :]
        acc0 = jnp.zeros((MB, EC, N), _F32)
        acc1 = jnp.zeros((MB, EC, N), _F32)
        for c in range(32):
            slab = jnp.maximum(ea_c[:, :, c:c + 1] + xaT3[:, c:c + 1, :],
                               0.0)
            if c % 2 == 0:
                acc0 = acc0 + slab * w2[c:c + 1, 0:1]
            else:
                acc1 = acc1 + slab * w2[c:c + 1, 0:1]
        chunks.append(acc0 + acc1)
    a3 = jnp.concatenate(chunks, axis=1) + ba2_ref[...][None]
    attn = jnp.maximum(a3, 0.0)

    logits = attn * H3
    m = jnp.max(logits, axis=2, keepdims=True)
    p = jnp.exp(logits - m)
    p = p / jnp.sum(p, axis=2, keepdims=True)
    Hw = p * H3
    m1f = jnp.concatenate([
        jax.lax.dot_general(
            H3[b],
            jnp.dot(Hw[b], xf[b * N:(b + 1) * N], preferred_element_type=_F32),
            (((0,), (0,)), ((), ())), preferred_element_type=_F32)
        for b in range(MB)], axis=0)

    We1 = We1_ref[...]
    sc = 1.0 / N
    h1 = jnp.maximum(
        (jnp.dot(m1f, We1[:HID], preferred_element_type=_F32)
         + jnp.dot(aff, We1[HID:], preferred_element_type=_F32)) * sc
        + be1_ref[...], 0.0)
    outf = jnp.maximum(jnp.dot(h1, We2_ref[...], preferred_element_type=_F32)
                       + be2_ref[...], 0.0)

    out_ref[...] = outf.reshape(MB, N, HID)
    H_ref[...] = H3


def _full(shape):
    nd = len(shape)
    return pl.BlockSpec(shape, lambda b, _nd=nd: (0,) * _nd)


def kernel(hidden_state, We, be, Wq, bq, W1, b1, W2, b2, Wa1, ba1, Wa2, ba2,
           We1, be1, We2, be2):
    be2d = be.reshape(1, HID)
    bq2d = bq.reshape(1, QK)
    b12d = b1.reshape(1, 2 * HID)
    b22d = b2.reshape(1, HID)
    ba12d = ba1.reshape(1, 32)
    ba22d = ba2.reshape(1, 1)
    be12d = be1.reshape(1, 2 * HID)
    be22d = be2.reshape(1, HID)

    out, H = pl.pallas_call(
        _hyper_body,
        grid=(B // MB,),
        in_specs=[
            pl.BlockSpec((MB, N, OBS), lambda b: (b, 0, 0)),
            _full((OBS, HID)), _full((1, HID)),
            _full((OBS, QK)), _full((1, QK)),
            _full((HID, 2 * HID)), _full((1, 2 * HID)),
            _full((2 * HID, HID)), _full((1, HID)),
            _full((2 * HID, 32)), _full((1, 32)),
            _full((32, 1)), _full((1, 1)),
            _full((2 * HID, 2 * HID)), _full((1, 2 * HID)),
            _full((2 * HID, HID)), _full((1, HID)),
        ],
        out_specs=[
            pl.BlockSpec((MB, N, HID), lambda b: (b, 0, 0)),
            pl.BlockSpec((MB, N, N), lambda b: (b, 0, 0)),
        ],
        out_shape=[
            jax.ShapeDtypeStruct((B, N, HID), _F32),
            jax.ShapeDtypeStruct((B, N, N), _F32),
        ],
    )(hidden_state, We, be2d, Wq, bq2d, W1, b12d, W2, b22d,
      Wa1, ba12d, Wa2, ba22d, We1, be12d, We2, be22d)
    return out, H


# speedup vs baseline: 1.0146x; 1.0146x over previous
import math

import jax
import jax.numpy as jnp
from jax.experimental import pallas as pl
from jax.experimental.pallas import tpu as pltpu

B, N, OBS, HID, QK = 16, 128, 128, 64, 32
KTOP = N // 4
MB = 16
_INV_SQRT_QK = 1.0 / math.sqrt(QK)
_F32 = jnp.float32


def _hyper_body(hs_ref, We_ref, be_ref, Wq_ref, bq_ref, W1_ref, b1_ref,
                W2_ref, b2_ref, Wa1_ref, ba1_ref, Wa2_ref, ba2_ref,
                We1_ref, be1_ref, We2_ref, be2_ref, out_ref, H_ref):
    hsf = hs_ref[...].reshape(MB * N, OBS)

    aff = jnp.maximum(jnp.dot(hsf, We_ref[...], preferred_element_type=_F32)
                      + be_ref[...], 0.0)
    qf = jnp.dot(hsf, Wq_ref[...], preferred_element_type=_F32) + bq_ref[...]
    S3 = jnp.stack([
        jax.lax.dot_general(qf[b * N:(b + 1) * N], qf[b * N:(b + 1) * N],
                            (((1,), (1,)), ((), ())),
                            preferred_element_type=_F32)
        for b in range(MB)]) * _INV_SQRT_QK

    lane_iota = jax.lax.broadcasted_iota(jnp.int32, (1, 1, N), 2)
    Ss = S3
    for sz_log in range(1, 8):
        sz = 1 << sz_log
        for st_log in range(sz_log - 1, -1, -1):
            st = 1 << st_log
            lower_m = (lane_iota & st) == 0
            keep_m = jnp.logical_not(
                jnp.logical_xor(lower_m, (lane_iota & sz) == 0))
            partner = jnp.where(lower_m, jnp.roll(Ss, -st, axis=2),
                                jnp.roll(Ss, st, axis=2))
            Ss = jnp.where(keep_m, jnp.minimum(Ss, partner),
                           jnp.maximum(Ss, partner))
    thr = Ss[:, :, N - KTOP:N - KTOP + 1]
    gtf = (S3 > thr).astype(_F32).reshape(MB * N, N)
    eqf = (S3 == thr).astype(_F32).reshape(MB * N, N)
    col_i = jax.lax.broadcasted_iota(jnp.int32, (N, N), 1)
    ones_nn = jnp.ones((N, N), _F32)
    lt_mat = (jax.lax.broadcasted_iota(jnp.int32, (N, N), 0)
              <= col_i).astype(_F32)
    cntf = (jnp.dot(gtf, ones_nn, preferred_element_type=_F32)
            + jnp.dot(eqf, lt_mat, preferred_element_type=_F32))
    Hf = jnp.where((gtf > 0.0) | ((eqf > 0.0) & (cntf <= float(KTOP))),
                   1.0, 0.0)
    H3 = Hf.reshape(MB, N, N)

    xf = jnp.maximum(jnp.dot(aff, W1_ref[...], preferred_element_type=_F32)
                     + b1_ref[...], 0.0)
    xf = jnp.maximum(jnp.dot(xf, W2_ref[...], preferred_element_type=_F32)
                     + b2_ref[...], 0.0)

    e0f = jnp.concatenate([
        jnp.dot(H3[b], xf[b * N:(b + 1) * N], preferred_element_type=_F32)
        for b in range(MB)], axis=0)

    Wa1 = Wa1_ref[...]
    eaf = (jnp.dot(e0f, Wa1[HID:], preferred_element_type=_F32)
           + ba1_ref[...])
    ea3 = eaf.reshape(MB, N, 32)
    xaT3 = jnp.stack([
        jax.lax.dot_general(Wa1[:HID], xf[b * N:(b + 1) * N],
                            (((0,), (1,)), ((), ())),
                            preferred_element_type=_F32)
        for b in range(MB)])

    w2 = Wa2_ref[...]
    accs = [jnp.zeros((MB, N, N), _F32) for _ in range(4)]
    for c in range(32):
        slab = jnp.maximum(ea3[:, :, c:c + 1] + xaT3[:, c:c + 1, :], 0.0)
        accs[c % 4] = accs[c % 4] + slab * w2[c:c + 1, 0:1]
    a3 = (accs[0] + accs[1]) + (accs[2] + accs[3]) + ba2_ref[...][None]
    attn = jnp.maximum(a3, 0.0)

    logits = attn * H3
    m = jnp.max(logits, axis=2, keepdims=True)
    p = jnp.exp(logits - m)
    denom = jnp.dot(p.reshape(MB * N, N), ones_nn,
                    preferred_element_type=_F32).reshape(MB, N, N)
    p = p / denom
    Hw = p * H3
    m1f = jnp.concatenate([
        jax.lax.dot_general(
            H3[b],
            jnp.dot(Hw[b], xf[b * N:(b + 1) * N], preferred_element_type=_F32),
            (((0,), (0,)), ((), ())), preferred_element_type=_F32)
        for b in range(MB)], axis=0)

    We1 = We1_ref[...]
    sc = 1.0 / N
    h1 = jnp.maximum(
        (jnp.dot(m1f, We1[:HID], preferred_element_type=_F32)
         + jnp.dot(aff, We1[HID:], preferred_element_type=_F32)) * sc
        + be1_ref[...], 0.0)
    outf = jnp.maximum(jnp.dot(h1, We2_ref[...], preferred_element_type=_F32)
                       + be2_ref[...], 0.0)

    out_ref[...] = outf.reshape(MB, N, HID)
    H_ref[...] = H3


def _full(shape):
    nd = len(shape)
    return pl.BlockSpec(shape, lambda b, _nd=nd: (0,) * _nd)


def kernel(hidden_state, We, be, Wq, bq, W1, b1, W2, b2, Wa1, ba1, Wa2, ba2,
           We1, be1, We2, be2):
    be2d = be.reshape(1, HID)
    bq2d = bq.reshape(1, QK)
    b12d = b1.reshape(1, 2 * HID)
    b22d = b2.reshape(1, HID)
    ba12d = ba1.reshape(1, 32)
    ba22d = ba2.reshape(1, 1)
    be12d = be1.reshape(1, 2 * HID)
    be22d = be2.reshape(1, HID)

    out, H = pl.pallas_call(
        _hyper_body,
        grid=(B // MB,),
        in_specs=[
            pl.BlockSpec((MB, N, OBS), lambda b: (b, 0, 0)),
            _full((OBS, HID)), _full((1, HID)),
            _full((OBS, QK)), _full((1, QK)),
            _full((HID, 2 * HID)), _full((1, 2 * HID)),
            _full((2 * HID, HID)), _full((1, HID)),
            _full((2 * HID, 32)), _full((1, 32)),
            _full((32, 1)), _full((1, 1)),
            _full((2 * HID, 2 * HID)), _full((1, 2 * HID)),
            _full((2 * HID, HID)), _full((1, HID)),
        ],
        out_specs=[
            pl.BlockSpec((MB, N, HID), lambda b: (b, 0, 0)),
            pl.BlockSpec((MB, N, N), lambda b: (b, 0, 0)),
        ],
        out_shape=[
            jax.ShapeDtypeStruct((B, N, HID), _F32),
            jax.ShapeDtypeStruct((B, N, N), _F32),
        ],
    )(hidden_state, We, be2d, Wq, bq2d, W1, b12d, W2, b22d,
      Wa1, ba12d, Wa2, ba22d, We1, be12d, We2, be22d)
    return out, H


# speedup vs baseline: 1.2641x; 1.2459x over previous
import math

import jax
import jax.numpy as jnp
from jax.experimental import pallas as pl
from jax.experimental.pallas import tpu as pltpu

B, N, OBS, HID, QK = 16, 128, 128, 64, 32
KTOP = N // 4
MB = 16
_INV_SQRT_QK = 1.0 / math.sqrt(QK)
_F32 = jnp.float32


def _hyper_body(hs_ref, We_ref, be_ref, Wq_ref, bq_ref, W1_ref, b1_ref,
                W2_ref, b2_ref, Wa1_ref, ba1_ref, Wa2_ref, ba2_ref,
                We1_ref, be1_ref, We2_ref, be2_ref, out_ref, H_ref):
    hsf = hs_ref[...].reshape(MB * N, OBS)

    aff = jnp.maximum(jnp.dot(hsf, We_ref[...], preferred_element_type=_F32)
                      + be_ref[...], 0.0)
    qf = jnp.dot(hsf, Wq_ref[...], preferred_element_type=_F32) + bq_ref[...]
    S3 = jnp.stack([
        jax.lax.dot_general(qf[b * N:(b + 1) * N], qf[b * N:(b + 1) * N],
                            (((1,), (1,)), ((), ())),
                            preferred_element_type=_F32)
        for b in range(MB)]) * _INV_SQRT_QK

    lane_iota = jax.lax.broadcasted_iota(jnp.int32, (1, 1, N), 2)
    lane_vec = jax.lax.iota(jnp.int32, N)
    Ss = S3
    for sz_log in range(1, 8):
        sz = 1 << sz_log
        for st_log in range(sz_log - 1, -1, -1):
            st = 1 << st_log
            lower_m = (lane_iota & st) == 0
            keep_m = jnp.logical_not(
                jnp.logical_xor(lower_m, (lane_iota & sz) == 0))
            partner = jnp.take_along_axis(
                Ss, jnp.broadcast_to(lane_iota ^ st, (MB, N, N)), axis=2)
            Ss = jnp.where(keep_m, jnp.minimum(Ss, partner),
                           jnp.maximum(Ss, partner))
    thr = Ss[:, :, N - KTOP:N - KTOP + 1]
    gtf = (S3 > thr).astype(_F32).reshape(MB * N, N)
    eqf = (S3 == thr).astype(_F32).reshape(MB * N, N)
    col_i = jax.lax.broadcasted_iota(jnp.int32, (N, N), 1)
    ones_nn = jnp.ones((N, N), _F32)
    lt_mat = (jax.lax.broadcasted_iota(jnp.int32, (N, N), 0)
              <= col_i).astype(_F32)
    cntf = (jnp.dot(gtf, ones_nn, preferred_element_type=_F32)
            + jnp.dot(eqf, lt_mat, preferred_element_type=_F32))
    Hf = jnp.where((gtf > 0.0) | ((eqf > 0.0) & (cntf <= float(KTOP))),
                   1.0, 0.0)
    H3 = Hf.reshape(MB, N, N)

    xf = jnp.maximum(jnp.dot(aff, W1_ref[...], preferred_element_type=_F32)
                     + b1_ref[...], 0.0)
    xf = jnp.maximum(jnp.dot(xf, W2_ref[...], preferred_element_type=_F32)
                     + b2_ref[...], 0.0)

    e0f = jnp.concatenate([
        jnp.dot(H3[b], xf[b * N:(b + 1) * N], preferred_element_type=_F32)
        for b in range(MB)], axis=0)

    Wa1 = Wa1_ref[...]
    eaf = (jnp.dot(e0f, Wa1[HID:], preferred_element_type=_F32)
           + ba1_ref[...])
    ea3 = eaf.reshape(MB, N, 32)
    xaT3 = jnp.stack([
        jax.lax.dot_general(Wa1[:HID], xf[b * N:(b + 1) * N],
                            (((0,), (1,)), ((), ())),
                            preferred_element_type=_F32)
        for b in range(MB)])

    w2 = Wa2_ref[...]
    accs = [jnp.zeros((MB, N, N), _F32) for _ in range(4)]
    for c in range(32):
        slab = jnp.maximum(ea3[:, :, c:c + 1] + xaT3[:, c:c + 1, :], 0.0)
        accs[c % 4] = accs[c % 4] + slab * w2[c:c + 1, 0:1]
    a3 = (accs[0] + accs[1]) + (accs[2] + accs[3]) + ba2_ref[...][None]
    attn = jnp.maximum(a3, 0.0)

    logits = attn * H3
    m = jnp.max(logits, axis=2, keepdims=True)
    p = jnp.exp(logits - m)
    p = p / jnp.sum(p, axis=2, keepdims=True)
    Hw = p * H3
    m1f = jnp.concatenate([
        jax.lax.dot_general(
            H3[b],
            jnp.dot(Hw[b], xf[b * N:(b + 1) * N], preferred_element_type=_F32),
            (((0,), (0,)), ((), ())), preferred_element_type=_F32)
        for b in range(MB)], axis=0)

    We1 = We1_ref[...]
    sc = 1.0 / N
    h1 = jnp.maximum(
        (jnp.dot(m1f, We1[:HID], preferred_element_type=_F32)
         + jnp.dot(aff, We1[HID:], preferred_element_type=_F32)) * sc
        + be1_ref[...], 0.0)
    outf = jnp.maximum(jnp.dot(h1, We2_ref[...], preferred_element_type=_F32)
                       + be2_ref[...], 0.0)

    out_ref[...] = outf.reshape(MB, N, HID)
    H_ref[...] = H3


def _full(shape):
    nd = len(shape)
    return pl.BlockSpec(shape, lambda b, _nd=nd: (0,) * _nd)


def kernel(hidden_state, We, be, Wq, bq, W1, b1, W2, b2, Wa1, ba1, Wa2, ba2,
           We1, be1, We2, be2):
    be2d = be.reshape(1, HID)
    bq2d = bq.reshape(1, QK)
    b12d = b1.reshape(1, 2 * HID)
    b22d = b2.reshape(1, HID)
    ba12d = ba1.reshape(1, 32)
    ba22d = ba2.reshape(1, 1)
    be12d = be1.reshape(1, 2 * HID)
    be22d = be2.reshape(1, HID)

    out, H = pl.pallas_call(
        _hyper_body,
        grid=(B // MB,),
        in_specs=[
            pl.BlockSpec((MB, N, OBS), lambda b: (b, 0, 0)),
            _full((OBS, HID)), _full((1, HID)),
            _full((OBS, QK)), _full((1, QK)),
            _full((HID, 2 * HID)), _full((1, 2 * HID)),
            _full((2 * HID, HID)), _full((1, HID)),
            _full((2 * HID, 32)), _full((1, 32)),
            _full((32, 1)), _full((1, 1)),
            _full((2 * HID, 2 * HID)), _full((1, 2 * HID)),
            _full((2 * HID, HID)), _full((1, HID)),
        ],
        out_specs=[
            pl.BlockSpec((MB, N, HID), lambda b: (b, 0, 0)),
            pl.BlockSpec((MB, N, N), lambda b: (b, 0, 0)),
        ],
        out_shape=[
            jax.ShapeDtypeStruct((B, N, HID), _F32),
            jax.ShapeDtypeStruct((B, N, N), _F32),
        ],
    )(hidden_state, We, be2d, Wq, bq2d, W1, b12d, W2, b22d,
      Wa1, ba12d, Wa2, ba22d, We1, be12d, We2, be22d)
    return out, H
